# baseline (speedup 1.0000x reference)
import functools

import jax
import jax.numpy as jnp
from jax import lax
from jax.experimental import pallas as pl
from jax.experimental.pallas import tpu as pltpu
from jax.experimental.pallas import tpu_sc as plsc

VOCAB = 1000000
SEQ = 200
BATCH = 4096
DIM = 64

B = BATCH * SEQ
NC, NS = 2, 16
NW = NC * NS
BPW = B // NW
CHUNK = 512
KSUB = CHUNK // 128
NCHUNK = BPW // CHUNK
LANES = 16
DSUB = DIM // LANES

_mesh = plsc.VectorSubcoreMesh(core_axis_name="c", subcore_axis_name="s")


@functools.partial(
    pl.kernel,
    mesh=_mesh,
    compiler_params=pltpu.CompilerParams(use_tc_tiling_on_sc=False),
    out_type=jax.ShapeDtypeStruct((B, DIM), jnp.float32),
    scratch_types=[
        pltpu.VMEM((SEQ, DIM), jnp.float32),
        pltpu.VMEM((BPW // 128, 128), jnp.int32),
        pltpu.VMEM((CHUNK, DIM), jnp.float32),
        pltpu.SemaphoreType.DMA,
    ],
)
def _embed(idx_hbm, table_hbm, pos_hbm, out_hbm, pos_v, idx_v, rows_v, sem):
    wid = lax.axis_index("s") * NC + lax.axis_index("c")
    base = wid * BPW
    pltpu.sync_copy(pos_hbm, pos_v)
    pltpu.sync_copy(idx_hbm.at[pl.ds(wid * (BPW // 128), BPW // 128)], idx_v)

    def chunk_body(g, carry):
        gbase = base + g * CHUNK
        cps = [
            pltpu.async_copy(
                table_hbm.at[idx_v.at[g * KSUB + j]],
                rows_v.at[pl.ds(j * 128, 128)],
                sem,
            )
            for j in range(KSUB)
        ]
        for cp in cps:
            cp.wait()
        p0 = lax.rem(gbase, SEQ)

        def row_body(r, c2):
            p = lax.rem(p0 + r, SEQ)
            for d in range(DSUB):
                sl = pl.ds(d * LANES, LANES)
                rows_v[r, sl] = rows_v[r, sl] + pos_v[p, sl]
            return c2

        lax.fori_loop(0, CHUNK, row_body, 0)
        pltpu.sync_copy(rows_v, out_hbm.at[pl.ds(gbase, CHUNK)])
        return carry

    lax.fori_loop(0, NCHUNK, chunk_body, 0)


def kernel(inputs, token_table, pos_table):
    idx = inputs.reshape(-1).astype(jnp.int32).reshape(B // 128, 128)
    out = _embed(idx, token_table, pos_table)
    return out.reshape(BATCH, SEQ, DIM)
